# baseline (speedup 1.0000x reference)
import jax
import jax.numpy as jnp
from jax.experimental import pallas as pl


_BS = 256


def _add_body(x_ref, pos_ref, out_ref):
    out_ref[...] = x_ref[...] + pos_ref[...][None]


def kernel(x, pos_table):
    batch, seq, d = x.shape
    return pl.pallas_call(
        _add_body,
        grid=(seq // _BS,),
        in_specs=[
            pl.BlockSpec((batch, _BS, d), lambda s: (0, s, 0)),
            pl.BlockSpec((_BS, d), lambda s: (s, 0)),
        ],
        out_specs=pl.BlockSpec((batch, _BS, d), lambda s: (0, s, 0)),
        out_shape=jax.ShapeDtypeStruct(x.shape, x.dtype),
    )(x, pos_table)
